# baseline (speedup 1.0000x reference)
import functools

import jax
import jax.numpy as jnp
from jax import lax
from jax.experimental import pallas as pl
from jax.experimental.pallas import tpu as pltpu
from jax.experimental.pallas import tpu_sc as plsc

N = 10000
E = 160000
DH = 100
DF = 4
LEVELS = 3
NUM_ROUNDS = 2

DP = 128
NP = 10240
BLK = 256
NC = 2
NS = 16
NW = NC * NS
K = 128
EPAD = 163840
EW = EPAD // NW
NB = EW // K
RPT = NP // NS
DUMP_ROW = N + 64


def _wspec():
    return pl.BlockSpec((DP, DP), lambda i: (0, 0))


def _bspec():
    return pl.BlockSpec((8, DP), lambda i: (0, 0))


def _rspec():
    return pl.BlockSpec((BLK, DP), lambda i: (i, 0))


def _mlp2_body(ns_ref, w1_ref, b1_ref, w2_ref, b2_ref, out_ref):
    h = jnp.maximum(
        jnp.dot(ns_ref[...], w1_ref[...], preferred_element_type=jnp.float32)
        + b1_ref[0:1, :], 0.0)
    out_ref[...] = (
        jnp.dot(h, w2_ref[...], preferred_element_type=jnp.float32)
        + b2_ref[0:1, :])


def _mlp2_tc(ns, w1, b1, w2, b2):
    return pl.pallas_call(
        _mlp2_body,
        grid=(NP // BLK,),
        in_specs=[_rspec(), _wspec(), _bspec(), _wspec(), _bspec()],
        out_specs=_rspec(),
        out_shape=jax.ShapeDtypeStruct((NP, DP), jnp.float32),
    )(ns, w1, b1, w2, b2)


def _affine_body(x_ref, w_ref, b_ref, out_ref):
    out_ref[...] = (
        jnp.dot(x_ref[...], w_ref[...], preferred_element_type=jnp.float32)
        + b_ref[0:1, :])


def _affine_tc(x, w, b):
    return pl.pallas_call(
        _affine_body,
        grid=(NP // BLK,),
        in_specs=[_rspec(), _wspec(), _bspec()],
        out_specs=_rspec(),
        out_shape=jax.ShapeDtypeStruct((NP, DP), jnp.float32),
    )(x, w, b)


def _update_body(lval, s_ref, x_ref, ns_ref,
                 lev_ref, pw1, pb1, pw2, pb2,
                 wir, bir, wiz, biz, win, bin_,
                 whr, bhr, whz, bhz, whn, bhn, out_ref):
    f32 = jnp.float32
    s = s_ref[...]
    h = jnp.maximum(
        jnp.dot(s, pw1[...], preferred_element_type=f32) + pb1[0:1, :], 0.0)
    msg = jnp.dot(h, pw2[...], preferred_element_type=f32) + pb2[0:1, :]
    x = x_ref[...]
    gr = (jnp.dot(x, wir[...], preferred_element_type=f32) + bir[0:1, :]
          + jnp.dot(msg, whr[...], preferred_element_type=f32) + bhr[0:1, :])
    gz = (jnp.dot(x, wiz[...], preferred_element_type=f32) + biz[0:1, :]
          + jnp.dot(msg, whz[...], preferred_element_type=f32) + bhz[0:1, :])
    r = jax.nn.sigmoid(gr)
    z = jax.nn.sigmoid(gz)
    hn = jnp.dot(msg, whn[...], preferred_element_type=f32) + bhn[0:1, :]
    gn = jnp.dot(x, win[...], preferred_element_type=f32) + bin_[0:1, :] + r * hn
    n = jnp.tanh(gn)
    nh = (1.0 - z) * n + z * msg
    mask = lev_ref[...] == lval
    out_ref[...] = jnp.where(mask, nh, ns_ref[...])


def _update_tc(lval, s, x, ns, lev, weights):
    return pl.pallas_call(
        functools.partial(_update_body, lval),
        grid=(NP // BLK,),
        in_specs=(
            [_rspec(), _rspec(), _rspec(), _rspec()]
            + [_wspec(), _bspec(), _wspec(), _bspec()]
            + [_wspec(), _bspec()] * 6),
        out_specs=_rspec(),
        out_shape=jax.ShapeDtypeStruct((NP, DP), jnp.float32),
    )(s, x, ns, lev, *weights)


def _cls_body(ns_ref, w1_ref, b1_ref, w2_ref, b2_ref, out_ref):
    h = jnp.maximum(
        jnp.dot(ns_ref[...], w1_ref[...], preferred_element_type=jnp.float32)
        + b1_ref[0:1, :], 0.0)
    out_ref[...] = jax.nn.sigmoid(
        jnp.dot(h, w2_ref[...], preferred_element_type=jnp.float32)
        + b2_ref[0:1, :])


def _cls_tc(ns, w1, b1, w2, b2):
    return pl.pallas_call(
        _cls_body,
        grid=(NP // BLK,),
        in_specs=[_rspec(), _wspec(), _bspec(), _wspec(), _bspec()],
        out_specs=_rspec(),
        out_shape=jax.ShapeDtypeStruct((NP, DP), jnp.float32),
    )(ns, w1, b1, w2, b2)


NQ = 4
QR = NP // NQ
EPT = EPAD // NS
NBT = EPT // K
RPT = QR // NS
NBUF = 5
AHEAD = 3
CH = 5
CB = NBT // CH


def _seg_sum_body(table, gidx, sidxq, out, idxg_c, idxs_c, rows, acc, gsem,
                  ssem):
    c = lax.axis_index("c")
    s = lax.axis_index("s")
    zero = jnp.zeros((16,), jnp.float32)
    base = s * NBT
    a0 = s * RPT

    def fire_gather(t, i):
        return pltpu.async_copy(table.at[idxg_c.at[t]], rows[i], gsem)

    def fire_scatter(t, i):
        return pltpu.async_copy(rows[i], acc.at[idxs_c.at[t]], ssem,
                                add=True)

    for qi in range(NQ // NC):
        qq = c * (NQ // NC) + qi

        def zrow(i, carry):
            for j in range(DP // 16):
                rows[0][i, pl.ds(j * 16, 16)] = zero
            return carry

        lax.fori_loop(0, K, zrow, 0)
        pltpu.sync_copy(rows[0], acc.at[pl.ds(a0, K)])
        pltpu.sync_copy(rows[0].at[pl.ds(0, RPT - K)],
                        acc.at[pl.ds(a0 + K, RPT - K)])
        plsc.subcore_barrier()

        def chunk(ch, carry):
            cb = base + ch * CB
            pltpu.sync_copy(gidx.at[pl.ds(cb, CB)], idxg_c)
            pltpu.sync_copy(sidxq.at[qq, pl.ds(cb, CB)], idxs_c)
            gd = {}
            sd = {}
            for t in range(AHEAD):
                gd[t] = fire_gather(t, t % NBUF)
            for t in range(CB):
                gd[t].wait()
                sd[t] = fire_scatter(t, t % NBUF)
                f = t + AHEAD
                if f < CB:
                    if f - NBUF >= 0:
                        sd[f - NBUF].wait()
                    gd[f] = fire_gather(f, f % NBUF)
            for t in range(CB - NBUF, CB):
                sd[t].wait()
            return carry

        lax.fori_loop(0, CH, chunk, 0)
        plsc.subcore_barrier()

        pltpu.sync_copy(acc.at[pl.ds(a0, K)], rows[0])
        pltpu.sync_copy(rows[0], out.at[qq, pl.ds(a0, K)])
        pltpu.sync_copy(acc.at[pl.ds(a0 + K, RPT - K)],
                        rows[1].at[pl.ds(0, RPT - K)])
        pltpu.sync_copy(rows[1].at[pl.ds(0, RPT - K)],
                        out.at[qq, pl.ds(a0 + K, RPT - K)])


@functools.cache
def _seg_sum_call():
    return pl.kernel(
        _seg_sum_body,
        out_type=jax.ShapeDtypeStruct((NQ, QR, DP), jnp.float32),
        mesh=plsc.VectorSubcoreMesh(
            core_axis_name="c", subcore_axis_name="s",
            num_cores=NC, num_subcores=NS),
        scratch_types=[
            pltpu.VMEM((CB, K), jnp.int32),
            pltpu.VMEM((CB, K), jnp.int32),
            [pltpu.VMEM((K, DP), jnp.float32)] * NBUF,
            pltpu.VMEM_SHARED((QR + 8, DP), jnp.float32),
            pltpu.SemaphoreType.DMA,
            pltpu.SemaphoreType.DMA,
        ],
    )


def _seg_sum(table, gidx, sidxq):
    out = _seg_sum_call()(table.reshape(NP, DP),
                          gidx.reshape(NBT * NS, K), sidxq)
    return out.reshape(NP, DP)


def _pad_w(w):
    out = jnp.zeros((DP, DP), jnp.float32)
    return out.at[:w.shape[0], :w.shape[1]].set(w.astype(jnp.float32))


def _pad_b(b):
    out = jnp.zeros((8, DP), jnp.float32)
    return out.at[0, :b.shape[0]].set(b.astype(jnp.float32))


def _gru_weights(Wih, bih, Whh, bhh):
    ws = []
    for g in range(3):
        ws.append(_pad_w(Wih[g * DH:(g + 1) * DH, :].T))
        ws.append(_pad_b(bih[g * DH:(g + 1) * DH]))
    hs = []
    for g in range(3):
        hs.append(_pad_w(Whh[g * DH:(g + 1) * DH, :].T))
        hs.append(_pad_b(bhh[g * DH:(g + 1) * DH]))
    return tuple(ws) + tuple(hs)


def kernel(x, edge_index, forward_level, backward_level, forward_index,
           backward_index, pf_W1, pf_b1, pf_W2, pf_b2, pf_pW1, pf_pb1,
           pf_pW2, pf_pb2, pb_W1, pb_b1, pb_W2, pb_b2, pb_pW1, pb_pb1,
           pb_pW2, pb_pb2, gf_Wih, gf_bih, gf_Whh, gf_bhh, gb_Wih, gb_bih,
           gb_Whh, gb_bhh, proj_W, proj_b, cls_W1, cls_b1, cls_W2, cls_b2):
    f32 = jnp.float32

    pre_f = (_pad_w(pf_W1), _pad_b(pf_b1), _pad_w(pf_W2), _pad_b(pf_b2))
    post_f = (_pad_w(pf_pW1), _pad_b(pf_pb1), _pad_w(pf_pW2), _pad_b(pf_pb2))
    pre_b = (_pad_w(pb_W1), _pad_b(pb_b1), _pad_w(pb_W2), _pad_b(pb_b2))
    post_b = (_pad_w(pb_pW1), _pad_b(pb_pb1), _pad_w(pb_pW2), _pad_b(pb_pb2))
    gru_f = _gru_weights(gf_Wih, gf_bih, gf_Whh, gf_bhh)
    gru_b = _gru_weights(gb_Wih, gb_bih, gb_Whh, gb_bhh)
    proj_Wp, proj_bp = _pad_w(proj_W), _pad_b(proj_b)
    cls = (_pad_w(cls_W1), _pad_b(cls_b1), _pad_w(cls_W2), _pad_b(cls_b2))

    x_pad = jnp.zeros((NP, DP), f32).at[:N, :DF].set(x.astype(f32))
    lev_f = jnp.full((NP, DP), -1, jnp.int32).at[:N].set(
        jnp.broadcast_to(forward_level[:, None], (N, DP)))
    lev_b = jnp.full((NP, DP), -1, jnp.int32).at[:N].set(
        jnp.broadcast_to(backward_level[:, None], (N, DP)))

    src = edge_index[0]
    dst = edge_index[1]
    pad_g = jnp.zeros((EPAD - E,), jnp.int32)
    pad_s = jnp.full((EPAD - E,), DUMP_ROW, jnp.int32)
    gidx_f = jnp.concatenate([src, pad_g])
    sidx_f = jnp.concatenate([dst, pad_s])
    gidx_b = jnp.concatenate([dst, pad_g])
    sidx_b = jnp.concatenate([src, pad_s])

    def quarter_idx(sidx):
        q = jnp.arange(NQ, dtype=jnp.int32)[:, None]
        v = sidx[None, :] - q * QR
        ok = (v >= 0) & (v < QR)
        return jnp.where(ok, v, QR).reshape(NQ, NBT * NS, K)

    sidxq_f = quarter_idx(sidx_f)
    sidxq_b = quarter_idx(sidx_b)

    ns = jnp.zeros((NP, DP), f32)
    xc = x_pad
    for rnd in range(NUM_ROUNDS):
        if rnd > 0:
            xc = _affine_tc(ns, proj_Wp, proj_bp)
        for lev, gi, si, pre, post, gru in (
                (lev_f, gidx_f, sidxq_f, pre_f, post_f, gru_f),
                (lev_b, gidx_b, sidxq_b, pre_b, post_b, gru_b)):
            for l in range(1, LEVELS):
                table = _mlp2_tc(ns, *pre)
                s = _seg_sum(table, gi, si)
                ns = _update_tc(l, s, xc, ns, lev, post + gru)
    out = _cls_tc(ns, *cls)
    return out[:N, :1]


# speedup vs baseline: 1.9167x; 1.9167x over previous
import functools

import jax
import jax.numpy as jnp
from jax import lax
from jax.experimental import pallas as pl
from jax.experimental.pallas import tpu as pltpu
from jax.experimental.pallas import tpu_sc as plsc

N = 10000
E = 160000
DH = 100
DF = 4
LEVELS = 3
NUM_ROUNDS = 2

DP = 128
NP = 10240
BLK = 256
NC = 2
NS = 16
NW = NC * NS
K = 128
EPAD = 163840
EW = EPAD // NW
NB = EW // K
RPT = NP // NS
DUMP_ROW = N + 64


def _wspec():
    return pl.BlockSpec((DP, DP), lambda i: (0, 0))


def _bspec():
    return pl.BlockSpec((8, DP), lambda i: (0, 0))


def _rspec():
    return pl.BlockSpec((BLK, DP), lambda i: (i, 0))


def _mlp2_body(ns_ref, w1_ref, b1_ref, w2_ref, b2_ref, out_ref):
    h = jnp.maximum(
        jnp.dot(ns_ref[...], w1_ref[...], preferred_element_type=jnp.float32)
        + b1_ref[0:1, :], 0.0)
    out_ref[...] = (
        jnp.dot(h, w2_ref[...], preferred_element_type=jnp.float32)
        + b2_ref[0:1, :])


def _mlp2_tc(ns, w1, b1, w2, b2):
    return pl.pallas_call(
        _mlp2_body,
        grid=(NP // BLK,),
        in_specs=[_rspec(), _wspec(), _bspec(), _wspec(), _bspec()],
        out_specs=_rspec(),
        out_shape=jax.ShapeDtypeStruct((NP, DP), jnp.float32),
    )(ns, w1, b1, w2, b2)


def _affine_body(x_ref, w_ref, b_ref, out_ref):
    out_ref[...] = (
        jnp.dot(x_ref[...], w_ref[...], preferred_element_type=jnp.float32)
        + b_ref[0:1, :])


def _affine_tc(x, w, b):
    return pl.pallas_call(
        _affine_body,
        grid=(NP // BLK,),
        in_specs=[_rspec(), _wspec(), _bspec()],
        out_specs=_rspec(),
        out_shape=jax.ShapeDtypeStruct((NP, DP), jnp.float32),
    )(x, w, b)


def _update_body(lval, s_ref, x_ref, ns_ref,
                 lev_ref, pw1, pb1, pw2, pb2,
                 wir, bir, wiz, biz, win, bin_,
                 whr, bhr, whz, bhz, whn, bhn, out_ref):
    f32 = jnp.float32
    s = s_ref[...]
    h = jnp.maximum(
        jnp.dot(s, pw1[...], preferred_element_type=f32) + pb1[0:1, :], 0.0)
    msg = jnp.dot(h, pw2[...], preferred_element_type=f32) + pb2[0:1, :]
    x = x_ref[...]
    gr = (jnp.dot(x, wir[...], preferred_element_type=f32) + bir[0:1, :]
          + jnp.dot(msg, whr[...], preferred_element_type=f32) + bhr[0:1, :])
    gz = (jnp.dot(x, wiz[...], preferred_element_type=f32) + biz[0:1, :]
          + jnp.dot(msg, whz[...], preferred_element_type=f32) + bhz[0:1, :])
    r = jax.nn.sigmoid(gr)
    z = jax.nn.sigmoid(gz)
    hn = jnp.dot(msg, whn[...], preferred_element_type=f32) + bhn[0:1, :]
    gn = jnp.dot(x, win[...], preferred_element_type=f32) + bin_[0:1, :] + r * hn
    n = jnp.tanh(gn)
    nh = (1.0 - z) * n + z * msg
    mask = lev_ref[...] == lval
    out_ref[...] = jnp.where(mask, nh, ns_ref[...])


def _update_tc(lval, s, x, ns, lev, weights):
    return pl.pallas_call(
        functools.partial(_update_body, lval),
        grid=(NP // BLK,),
        in_specs=(
            [_rspec(), _rspec(), _rspec(), _rspec()]
            + [_wspec(), _bspec(), _wspec(), _bspec()]
            + [_wspec(), _bspec()] * 6),
        out_specs=_rspec(),
        out_shape=jax.ShapeDtypeStruct((NP, DP), jnp.float32),
    )(s, x, ns, lev, *weights)


def _cls_body(ns_ref, w1_ref, b1_ref, w2_ref, b2_ref, out_ref):
    h = jnp.maximum(
        jnp.dot(ns_ref[...], w1_ref[...], preferred_element_type=jnp.float32)
        + b1_ref[0:1, :], 0.0)
    out_ref[...] = jax.nn.sigmoid(
        jnp.dot(h, w2_ref[...], preferred_element_type=jnp.float32)
        + b2_ref[0:1, :])


def _cls_tc(ns, w1, b1, w2, b2):
    return pl.pallas_call(
        _cls_body,
        grid=(NP // BLK,),
        in_specs=[_rspec(), _wspec(), _bspec(), _wspec(), _bspec()],
        out_specs=_rspec(),
        out_shape=jax.ShapeDtypeStruct((NP, DP), jnp.float32),
    )(ns, w1, b1, w2, b2)


NQ = 2
QR = NP // NQ
EPT = EPAD // NS
NBT = EPT // K
RPT = QR // NS
NBUF = 5
AHEAD = 3
CH = 5
CB = NBT // CH


def _seg_sum_body(table, gidx, sidxq, out, idxg_c, idxs_c, rows, acc, gsem,
                  ssem):
    c = lax.axis_index("c")
    s = lax.axis_index("s")
    zero = jnp.zeros((16,), jnp.float32)
    base = s * NBT
    a0 = s * RPT

    def fire_gather(t, i):
        return pltpu.async_copy(table.at[idxg_c.at[t]], rows[i], gsem)

    def fire_scatter(t, i):
        return pltpu.async_copy(rows[i], acc.at[idxs_c.at[t]], ssem,
                                add=True)

    for qi in range(NQ // NC):
        qq = c * (NQ // NC) + qi

        def zrow(i, carry):
            for j in range(DP // 16):
                rows[0][i, pl.ds(j * 16, 16)] = zero
            return carry

        lax.fori_loop(0, K, zrow, 0)
        for off in range(0, RPT, K):
            sz = min(K, RPT - off)
            pltpu.sync_copy(rows[0].at[pl.ds(0, sz)],
                            acc.at[pl.ds(a0 + off, sz)])
        plsc.subcore_barrier()

        def chunk(ch, carry):
            cb = base + ch * CB
            pltpu.sync_copy(gidx.at[pl.ds(cb, CB)], idxg_c)
            pltpu.sync_copy(sidxq.at[qq, pl.ds(cb, CB)], idxs_c)
            gd = {}
            sd = {}
            for t in range(AHEAD):
                gd[t] = fire_gather(t, t % NBUF)
            for t in range(CB):
                gd[t].wait()
                sd[t] = fire_scatter(t, t % NBUF)
                f = t + AHEAD
                if f < CB:
                    if f - NBUF >= 0:
                        sd[f - NBUF].wait()
                    gd[f] = fire_gather(f, f % NBUF)
            for t in range(CB - NBUF, CB):
                sd[t].wait()
            return carry

        lax.fori_loop(0, CH, chunk, 0)
        plsc.subcore_barrier()

        for off in range(0, RPT, K):
            sz = min(K, RPT - off)
            pltpu.sync_copy(acc.at[pl.ds(a0 + off, sz)],
                            rows[0].at[pl.ds(0, sz)])
            pltpu.sync_copy(rows[0].at[pl.ds(0, sz)],
                            out.at[qq, pl.ds(a0 + off, sz)])


@functools.cache
def _seg_sum_call():
    return pl.kernel(
        _seg_sum_body,
        out_type=jax.ShapeDtypeStruct((NQ, QR, DP), jnp.float32),
        mesh=plsc.VectorSubcoreMesh(
            core_axis_name="c", subcore_axis_name="s",
            num_cores=NC, num_subcores=NS),
        scratch_types=[
            pltpu.VMEM((CB, K), jnp.int32),
            pltpu.VMEM((CB, K), jnp.int32),
            [pltpu.VMEM((K, DP), jnp.float32)] * NBUF,
            pltpu.VMEM_SHARED((QR + 8, DP), jnp.float32),
            pltpu.SemaphoreType.DMA,
            pltpu.SemaphoreType.DMA,
        ],
    )


def _seg_sum(table, gidx, sidxq):
    out = _seg_sum_call()(table.reshape(NP, DP),
                          gidx.reshape(NBT * NS, K), sidxq)
    return out.reshape(NP, DP)


def _pad_w(w):
    out = jnp.zeros((DP, DP), jnp.float32)
    return out.at[:w.shape[0], :w.shape[1]].set(w.astype(jnp.float32))


def _pad_b(b):
    out = jnp.zeros((8, DP), jnp.float32)
    return out.at[0, :b.shape[0]].set(b.astype(jnp.float32))


def _gru_weights(Wih, bih, Whh, bhh):
    ws = []
    for g in range(3):
        ws.append(_pad_w(Wih[g * DH:(g + 1) * DH, :].T))
        ws.append(_pad_b(bih[g * DH:(g + 1) * DH]))
    hs = []
    for g in range(3):
        hs.append(_pad_w(Whh[g * DH:(g + 1) * DH, :].T))
        hs.append(_pad_b(bhh[g * DH:(g + 1) * DH]))
    return tuple(ws) + tuple(hs)


def kernel(x, edge_index, forward_level, backward_level, forward_index,
           backward_index, pf_W1, pf_b1, pf_W2, pf_b2, pf_pW1, pf_pb1,
           pf_pW2, pf_pb2, pb_W1, pb_b1, pb_W2, pb_b2, pb_pW1, pb_pb1,
           pb_pW2, pb_pb2, gf_Wih, gf_bih, gf_Whh, gf_bhh, gb_Wih, gb_bih,
           gb_Whh, gb_bhh, proj_W, proj_b, cls_W1, cls_b1, cls_W2, cls_b2):
    f32 = jnp.float32

    pre_f = (_pad_w(pf_W1), _pad_b(pf_b1), _pad_w(pf_W2), _pad_b(pf_b2))
    post_f = (_pad_w(pf_pW1), _pad_b(pf_pb1), _pad_w(pf_pW2), _pad_b(pf_pb2))
    pre_b = (_pad_w(pb_W1), _pad_b(pb_b1), _pad_w(pb_W2), _pad_b(pb_b2))
    post_b = (_pad_w(pb_pW1), _pad_b(pb_pb1), _pad_w(pb_pW2), _pad_b(pb_pb2))
    gru_f = _gru_weights(gf_Wih, gf_bih, gf_Whh, gf_bhh)
    gru_b = _gru_weights(gb_Wih, gb_bih, gb_Whh, gb_bhh)
    proj_Wp, proj_bp = _pad_w(proj_W), _pad_b(proj_b)
    cls = (_pad_w(cls_W1), _pad_b(cls_b1), _pad_w(cls_W2), _pad_b(cls_b2))

    x_pad = jnp.zeros((NP, DP), f32).at[:N, :DF].set(x.astype(f32))
    lev_f = jnp.full((NP, DP), -1, jnp.int32).at[:N].set(
        jnp.broadcast_to(forward_level[:, None], (N, DP)))
    lev_b = jnp.full((NP, DP), -1, jnp.int32).at[:N].set(
        jnp.broadcast_to(backward_level[:, None], (N, DP)))

    src = edge_index[0]
    dst = edge_index[1]
    pad_g = jnp.zeros((EPAD - E,), jnp.int32)
    pad_s = jnp.full((EPAD - E,), DUMP_ROW, jnp.int32)
    gidx_f = jnp.concatenate([src, pad_g])
    sidx_f = jnp.concatenate([dst, pad_s])
    gidx_b = jnp.concatenate([dst, pad_g])
    sidx_b = jnp.concatenate([src, pad_s])

    def quarter_idx(sidx):
        q = jnp.arange(NQ, dtype=jnp.int32)[:, None]
        v = sidx[None, :] - q * QR
        ok = (v >= 0) & (v < QR)
        return jnp.where(ok, v, QR).reshape(NQ, NBT * NS, K)

    sidxq_f = quarter_idx(sidx_f)
    sidxq_b = quarter_idx(sidx_b)

    ns = jnp.zeros((NP, DP), f32)
    xc = x_pad
    for rnd in range(NUM_ROUNDS):
        if rnd > 0:
            xc = _affine_tc(ns, proj_Wp, proj_bp)
        for lev, gi, si, pre, post, gru in (
                (lev_f, gidx_f, sidxq_f, pre_f, post_f, gru_f),
                (lev_b, gidx_b, sidxq_b, pre_b, post_b, gru_b)):
            for l in range(1, LEVELS):
                table = _mlp2_tc(ns, *pre)
                s = _seg_sum(table, gi, si)
                ns = _update_tc(l, s, xc, ns, lev, post + gru)
    out = _cls_tc(ns, *cls)
    return out[:N, :1]


# speedup vs baseline: 1.9744x; 1.0301x over previous
import functools

import jax
import jax.numpy as jnp
from jax import lax
from jax.experimental import pallas as pl
from jax.experimental.pallas import tpu as pltpu
from jax.experimental.pallas import tpu_sc as plsc

N = 10000
E = 160000
DH = 100
DF = 4
LEVELS = 3
NUM_ROUNDS = 2

DP = 128
NP = 10240
BLK = 256
NC = 2
NS = 16
NW = NC * NS
K = 128
EPAD = 163840
EW = EPAD // NW
NB = EW // K
RPT = NP // NS
DUMP_ROW = N + 64


def _wspec():
    return pl.BlockSpec((DP, DP), lambda i: (0, 0))


def _bspec():
    return pl.BlockSpec((8, DP), lambda i: (0, 0))


def _rspec():
    return pl.BlockSpec((BLK, DP), lambda i: (i, 0))


def _mlp2_body(ns_ref, w1_ref, b1_ref, w2_ref, b2_ref, out_ref):
    h = jnp.maximum(
        jnp.dot(ns_ref[...], w1_ref[...], preferred_element_type=jnp.float32)
        + b1_ref[0:1, :], 0.0)
    out_ref[...] = (
        jnp.dot(h, w2_ref[...], preferred_element_type=jnp.float32)
        + b2_ref[0:1, :])


def _mlp2_tc(ns, w1, b1, w2, b2):
    return pl.pallas_call(
        _mlp2_body,
        grid=(NP // BLK,),
        in_specs=[_rspec(), _wspec(), _bspec(), _wspec(), _bspec()],
        out_specs=_rspec(),
        out_shape=jax.ShapeDtypeStruct((NP, DP), jnp.float32),
    )(ns, w1, b1, w2, b2)


def _affine_body(x_ref, w_ref, b_ref, out_ref):
    out_ref[...] = (
        jnp.dot(x_ref[...], w_ref[...], preferred_element_type=jnp.float32)
        + b_ref[0:1, :])


def _affine_tc(x, w, b):
    return pl.pallas_call(
        _affine_body,
        grid=(NP // BLK,),
        in_specs=[_rspec(), _wspec(), _bspec()],
        out_specs=_rspec(),
        out_shape=jax.ShapeDtypeStruct((NP, DP), jnp.float32),
    )(x, w, b)


def _update_body(lval, s_ref, x_ref, ns_ref,
                 lev_ref, pw1, pb1, pw2, pb2,
                 wir, bir, wiz, biz, win, bin_,
                 whr, bhr, whz, bhz, whn, bhn,
                 nw1, nb1, nw2, nb2, out_ref, tab_ref):
    f32 = jnp.float32
    s = s_ref[...]
    h = jnp.maximum(
        jnp.dot(s, pw1[...], preferred_element_type=f32) + pb1[0:1, :], 0.0)
    msg = jnp.dot(h, pw2[...], preferred_element_type=f32) + pb2[0:1, :]
    x = x_ref[...]
    gr = (jnp.dot(x, wir[...], preferred_element_type=f32) + bir[0:1, :]
          + jnp.dot(msg, whr[...], preferred_element_type=f32) + bhr[0:1, :])
    gz = (jnp.dot(x, wiz[...], preferred_element_type=f32) + biz[0:1, :]
          + jnp.dot(msg, whz[...], preferred_element_type=f32) + bhz[0:1, :])
    r = jax.nn.sigmoid(gr)
    z = jax.nn.sigmoid(gz)
    hn = jnp.dot(msg, whn[...], preferred_element_type=f32) + bhn[0:1, :]
    gn = jnp.dot(x, win[...], preferred_element_type=f32) + bin_[0:1, :] + r * hn
    n = jnp.tanh(gn)
    nh = (1.0 - z) * n + z * msg
    mask = lev_ref[...] == lval
    ns_new = jnp.where(mask, nh, ns_ref[...])
    out_ref[...] = ns_new
    h2 = jnp.maximum(
        jnp.dot(ns_new, nw1[...], preferred_element_type=f32) + nb1[0:1, :],
        0.0)
    tab_ref[...] = (
        jnp.dot(h2, nw2[...], preferred_element_type=f32) + nb2[0:1, :])


def _update_tc(lval, s, x, ns, lev, weights, next_pre):
    return pl.pallas_call(
        functools.partial(_update_body, lval),
        grid=(NP // BLK,),
        in_specs=(
            [_rspec(), _rspec(), _rspec(), _rspec()]
            + [_wspec(), _bspec(), _wspec(), _bspec()]
            + [_wspec(), _bspec()] * 6
            + [_wspec(), _bspec(), _wspec(), _bspec()]),
        out_specs=(_rspec(), _rspec()),
        out_shape=(jax.ShapeDtypeStruct((NP, DP), jnp.float32),
                   jax.ShapeDtypeStruct((NP, DP), jnp.float32)),
    )(s, x, ns, lev, *weights, *next_pre)


def _cls_body(ns_ref, w1_ref, b1_ref, w2_ref, b2_ref, out_ref):
    h = jnp.maximum(
        jnp.dot(ns_ref[...], w1_ref[...], preferred_element_type=jnp.float32)
        + b1_ref[0:1, :], 0.0)
    out_ref[...] = jax.nn.sigmoid(
        jnp.dot(h, w2_ref[...], preferred_element_type=jnp.float32)
        + b2_ref[0:1, :])


def _cls_tc(ns, w1, b1, w2, b2):
    return pl.pallas_call(
        _cls_body,
        grid=(NP // BLK,),
        in_specs=[_rspec(), _wspec(), _bspec(), _wspec(), _bspec()],
        out_specs=_rspec(),
        out_shape=jax.ShapeDtypeStruct((NP, DP), jnp.float32),
    )(ns, w1, b1, w2, b2)


NQ = 2
QR = NP // NQ
EPT = EPAD // NS
RPT = QR // NS
NBUF = 5
AHEAD = 3
CB = 8
NR = EPAD // K


def _seg_sum_body(table, gidx, sidxq, out, idxg_c, idxs_c, rows,
                  acc, gsem, ssem):
    c = lax.axis_index("c")
    s = lax.axis_index("s")
    zero = jnp.zeros((16,), jnp.float32)
    a0 = s * RPT

    def fire_gather(t, i):
        return pltpu.async_copy(table.at[idxg_c.at[t]], rows[i], gsem)

    def fire_scatter(t, i):
        return pltpu.async_copy(rows[i], acc.at[idxs_c.at[t]], ssem,
                                add=True)

    nch = NR // NS // CB
    t0 = s * nch

    def zrow(i, carry):
        for j in range(DP // 16):
            rows[0][i, pl.ds(j * 16, 16)] = zero
        return carry

    lax.fori_loop(0, K, zrow, 0)
    for off in range(0, RPT, K):
        sz = min(K, RPT - off)
        pltpu.sync_copy(rows[0].at[pl.ds(0, sz)],
                        acc.at[pl.ds(a0 + off, sz)])
    plsc.subcore_barrier()

    def chunk(ch, carry):
        cb = (t0 + ch) * CB
        pltpu.sync_copy(gidx.at[pl.ds(cb, CB)], idxg_c)
        pltpu.sync_copy(sidxq.at[c, pl.ds(cb, CB)], idxs_c)
        gd = {}
        sd = {}
        for t in range(AHEAD):
            gd[t] = fire_gather(t, t % NBUF)
        for t in range(CB):
            gd[t].wait()
            sd[t] = fire_scatter(t, t % NBUF)
            f = t + AHEAD
            if f < CB:
                if f - NBUF >= 0:
                    sd[f - NBUF].wait()
                gd[f] = fire_gather(f, f % NBUF)
        for t in range(CB - NBUF, CB):
            sd[t].wait()
        return carry

    lax.fori_loop(0, nch, chunk, 0)
    plsc.subcore_barrier()

    for off in range(0, RPT, K):
        sz = min(K, RPT - off)
        pltpu.sync_copy(acc.at[pl.ds(a0 + off, sz)],
                        rows[0].at[pl.ds(0, sz)])
        pltpu.sync_copy(rows[0].at[pl.ds(0, sz)],
                        out.at[c, pl.ds(a0 + off, sz)])


@functools.cache
def _seg_sum_call():
    return pl.kernel(
        _seg_sum_body,
        out_type=jax.ShapeDtypeStruct((NQ, QR, DP), jnp.float32),
        mesh=plsc.VectorSubcoreMesh(
            core_axis_name="c", subcore_axis_name="s",
            num_cores=NC, num_subcores=NS),
        scratch_types=[
            pltpu.VMEM((CB, K), jnp.int32),
            pltpu.VMEM((CB, K), jnp.int32),
            [pltpu.VMEM((K, DP), jnp.float32)] * NBUF,
            pltpu.VMEM_SHARED((QR + 8, DP), jnp.float32),
            pltpu.SemaphoreType.DMA,
            pltpu.SemaphoreType.DMA,
        ],
    )


def _seg_sum(table, gidx, sidxq):
    out = _seg_sum_call()(table.reshape(NP, DP), gidx, sidxq)
    return out.reshape(NP, DP)


def _pad_w(w):
    out = jnp.zeros((DP, DP), jnp.float32)
    return out.at[:w.shape[0], :w.shape[1]].set(w.astype(jnp.float32))


def _pad_b(b):
    out = jnp.zeros((8, DP), jnp.float32)
    return out.at[0, :b.shape[0]].set(b.astype(jnp.float32))


def _gru_weights(Wih, bih, Whh, bhh):
    ws = []
    for g in range(3):
        ws.append(_pad_w(Wih[g * DH:(g + 1) * DH, :].T))
        ws.append(_pad_b(bih[g * DH:(g + 1) * DH]))
    hs = []
    for g in range(3):
        hs.append(_pad_w(Whh[g * DH:(g + 1) * DH, :].T))
        hs.append(_pad_b(bhh[g * DH:(g + 1) * DH]))
    return tuple(ws) + tuple(hs)


def kernel(x, edge_index, forward_level, backward_level, forward_index,
           backward_index, pf_W1, pf_b1, pf_W2, pf_b2, pf_pW1, pf_pb1,
           pf_pW2, pf_pb2, pb_W1, pb_b1, pb_W2, pb_b2, pb_pW1, pb_pb1,
           pb_pW2, pb_pb2, gf_Wih, gf_bih, gf_Whh, gf_bhh, gb_Wih, gb_bih,
           gb_Whh, gb_bhh, proj_W, proj_b, cls_W1, cls_b1, cls_W2, cls_b2):
    f32 = jnp.float32

    pre_f = (_pad_w(pf_W1), _pad_b(pf_b1), _pad_w(pf_W2), _pad_b(pf_b2))
    post_f = (_pad_w(pf_pW1), _pad_b(pf_pb1), _pad_w(pf_pW2), _pad_b(pf_pb2))
    pre_b = (_pad_w(pb_W1), _pad_b(pb_b1), _pad_w(pb_W2), _pad_b(pb_b2))
    post_b = (_pad_w(pb_pW1), _pad_b(pb_pb1), _pad_w(pb_pW2), _pad_b(pb_pb2))
    gru_f = _gru_weights(gf_Wih, gf_bih, gf_Whh, gf_bhh)
    gru_b = _gru_weights(gb_Wih, gb_bih, gb_Whh, gb_bhh)
    proj_Wp, proj_bp = _pad_w(proj_W), _pad_b(proj_b)
    cls = (_pad_w(cls_W1), _pad_b(cls_b1), _pad_w(cls_W2), _pad_b(cls_b2))

    x_pad = jnp.zeros((NP, DP), f32).at[:N, :DF].set(x.astype(f32))
    lev_f = jnp.full((NP, DP), -1, jnp.int32).at[:N].set(
        jnp.broadcast_to(forward_level[:, None], (N, DP)))
    lev_b = jnp.full((NP, DP), -1, jnp.int32).at[:N].set(
        jnp.broadcast_to(backward_level[:, None], (N, DP)))

    src = edge_index[0]
    dst = edge_index[1]
    pad_g = jnp.zeros((EPAD - E,), jnp.int32)
    pad_s = jnp.full((EPAD - E,), DUMP_ROW, jnp.int32)
    gidx_f = jnp.concatenate([src, pad_g])
    sidx_f = jnp.concatenate([dst, pad_s])
    gidx_b = jnp.concatenate([dst, pad_g])
    sidx_b = jnp.concatenate([src, pad_s])

    def half_planes(gidx, sidx):
        i32 = jnp.int32
        q = jnp.arange(NQ, dtype=i32)[:, None]
        v = sidx[None, :] - q * QR
        ok = (v >= 0) & (v < QR)
        sidxq = jnp.where(ok, v, QR).reshape(NQ, NR, K)
        return gidx.reshape(NR, K), sidxq

    gidx_f, sidxq_f = half_planes(gidx_f, sidx_f)
    gidx_b, sidxq_b = half_planes(gidx_b, sidx_b)

    layers = []
    for rnd in range(NUM_ROUNDS):
        for lev, gi, si, pre, post, gru in (
                (lev_f, gidx_f, sidxq_f, pre_f, post_f, gru_f),
                (lev_b, gidx_b, sidxq_b, pre_b, post_b, gru_b)):
            for l in range(1, LEVELS):
                layers.append((lev, gi, si, pre, post, gru, l))

    ns = jnp.zeros((NP, DP), f32)
    xc = x_pad
    table = _mlp2_tc(ns, *layers[0][3])
    for k, (lev, gi, si, pre, post, gru, l) in enumerate(layers):
        nxt = layers[k + 1][3] if k + 1 < len(layers) else pre
        s = _seg_sum(table, gi, si)
        ns, table = _update_tc(l, s, xc, ns, lev, post + gru, nxt)
        if k == len(layers) // 2 - 1:
            xc = _affine_tc(ns, proj_Wp, proj_bp)
    out = _cls_tc(ns, *cls)
    return out[:N, :1]
